# baseline (speedup 1.0000x reference)
import functools

import jax
import jax.numpy as jnp
from jax import lax
from jax.experimental import pallas as pl
from jax.experimental.pallas import tpu as pltpu
from jax.experimental.pallas import tpu_sc as plsc

_N = 100000
_D = 128
_NUM_CLASSES = 1000
_C = 400
_G = 80
_NCHUNKS = _N // _C
_NW = 32
_L = 16

_mesh = plsc.VectorSubcoreMesh(core_axis_name="c", subcore_axis_name="s")


@functools.partial(
    pl.kernel,
    mesh=_mesh,
    out_type=jax.ShapeDtypeStruct((_N, _D), jnp.float32),
    scratch_types=[
        pltpu.VMEM((_C,), jnp.int32),
        pltpu.VMEM((_C,), jnp.int32),
        pltpu.VMEM((_C,), jnp.int32),
        pltpu.VMEM((_C, _D), jnp.float32),
        pltpu.VMEM((_C, _D), jnp.float32),
        pltpu.SemaphoreType.DMA,
        pltpu.SemaphoreType.DMA,
    ],
)
def _mask_label_sc(x_hbm, y_hbm, m_hbm, tab_hbm, out_hbm,
                   y_v, m_v, idx_v, x_v, e_v, sem_x, sem_g):
    wid = lax.axis_index("s") * 2 + lax.axis_index("c")
    nch = (_NCHUNKS - wid + _NW - 1) // _NW

    def chunk_body(k, carry):
        base = (wid + k * _NW) * _C
        cp_x = pltpu.async_copy(x_hbm.at[pl.ds(base, _C)], x_v, sem_x)
        pltpu.sync_copy(y_hbm.at[pl.ds(base, _C)], y_v)
        pltpu.sync_copy(m_hbm.at[pl.ds(base, _C)], m_v)

        def sel_body(g, c2):
            s = pl.ds(g * _L, _L)
            yv = y_v[s]
            idx_v[s] = jnp.where(m_v[s] != 0, yv, yv + _NUM_CLASSES)
            return c2

        lax.fori_loop(0, _C // _L, sel_body, 0)

        cps = [
            pltpu.async_copy(tab_hbm.at[idx_v.at[pl.ds(j * _G, _G)]],
                             e_v.at[pl.ds(j * _G, _G)], sem_g)
            for j in range(_C // _G)
        ]
        cp_x.wait()
        for cp in cps:
            cp.wait()

        def add_body(r, c2):
            for j in range(_D // _L):
                s = pl.ds(j * _L, _L)
                x_v[r, s] = x_v[r, s] + e_v[r, s]
            return c2

        lax.fori_loop(0, _C, add_body, 0)

        pltpu.sync_copy(x_v, out_hbm.at[pl.ds(base, _C)])
        return carry

    lax.fori_loop(0, nch, chunk_body, 0)


def kernel(x, y, mask, emb_weight):
    table = jnp.concatenate(
        [emb_weight, jnp.zeros((_NUM_CLASSES, _D), jnp.float32)], axis=0)
    return _mask_label_sc(x, y, mask.astype(jnp.int32), table)


# speedup vs baseline: 1.2342x; 1.2342x over previous
import functools

import jax
import jax.numpy as jnp
from jax import lax
from jax.experimental import pallas as pl
from jax.experimental.pallas import tpu as pltpu
from jax.experimental.pallas import tpu_sc as plsc

_N = 100000
_D = 128
_NUM_CLASSES = 1000
_C = 400
_G = 80
_NCHUNKS = _N // _C
_NW = 32
_L = 16

_mesh = plsc.VectorSubcoreMesh(core_axis_name="c", subcore_axis_name="s")


@functools.partial(
    pl.kernel,
    mesh=_mesh,
    out_type=jax.ShapeDtypeStruct((_N, _D), jnp.float32),
    scratch_types=[
        pltpu.VMEM((_C,), jnp.int32),
        pltpu.VMEM((_C,), jnp.int32),
        pltpu.VMEM((_C,), jnp.int32),
        pltpu.VMEM((_C, _D), jnp.float32),
        pltpu.VMEM((_C, _D), jnp.float32),
        pltpu.SemaphoreType.DMA,
        pltpu.SemaphoreType.DMA,
    ],
)
def _mask_label_sc(x_hbm, y_hbm, m_hbm, tab_hbm, out_hbm,
                   y_v, m_v, idx_v, x_v, e_v, sem_x, sem_g):
    wid = lax.axis_index("s") * 2 + lax.axis_index("c")
    nch = (_NCHUNKS - wid + _NW - 1) // _NW

    def chunk_body(k, carry):
        base = (wid + k * _NW) * _C
        cp_x = pltpu.async_copy(x_hbm.at[pl.ds(base, _C)], x_v, sem_x)
        pltpu.sync_copy(y_hbm.at[pl.ds(base, _C)], y_v)
        pltpu.sync_copy(m_hbm.at[pl.ds(base, _C)], m_v)

        def sel_body(g, c2):
            s = pl.ds(g * _L, _L)
            yv = y_v[s]
            idx_v[s] = jnp.where(m_v[s] != 0, yv, yv + _NUM_CLASSES)
            return c2

        lax.fori_loop(0, _C // _L, sel_body, 0)

        cp_x.wait()
        cps = [
            pltpu.async_copy(tab_hbm.at[idx_v.at[pl.ds(j * _G, _G)]],
                             x_v.at[pl.ds(j * _G, _G)], sem_g, add=True)
            for j in range(_C // _G)
        ]
        for cp in cps:
            cp.wait()

        pltpu.sync_copy(x_v, out_hbm.at[pl.ds(base, _C)])
        return carry

    lax.fori_loop(0, nch, chunk_body, 0)


def kernel(x, y, mask, emb_weight):
    table = jnp.concatenate(
        [emb_weight, jnp.zeros((_NUM_CLASSES, _D), jnp.float32)], axis=0)
    return _mask_label_sc(x, y, mask.astype(jnp.int32), table)


# speedup vs baseline: 1.3258x; 1.0742x over previous
import functools

import jax
import jax.numpy as jnp
from jax import lax
from jax.experimental import pallas as pl
from jax.experimental.pallas import tpu as pltpu
from jax.experimental.pallas import tpu_sc as plsc

_N = 100000
_D = 128
_NUM_CLASSES = 1000
_C = 160
_G = 80
_NCHUNKS = _N // _C
_NW = 32
_NB = 4
_L = 16
_NGROUPS = (_NCHUNKS // _NW + _NB) // _NB

_mesh = plsc.VectorSubcoreMesh(core_axis_name="c", subcore_axis_name="s")


@functools.partial(
    pl.kernel,
    mesh=_mesh,
    out_type=jax.ShapeDtypeStruct((_N, _D), jnp.float32),
    scratch_types=[
        pltpu.VMEM((_NB * _C,), jnp.int32),
        pltpu.VMEM((_NB * _C,), jnp.int32),
        pltpu.VMEM((_NB * _C,), jnp.int32),
        pltpu.VMEM((_NB, _C, _D), jnp.float32),
        pltpu.SemaphoreType.DMA((_NB,)),
        pltpu.SemaphoreType.DMA((_NB,)),
        pltpu.SemaphoreType.DMA((_NB,)),
        pltpu.SemaphoreType.DMA((_NB,)),
    ],
)
def _mask_label_sc(x_hbm, y_hbm, m_hbm, tab_hbm, out_hbm,
                   y_v, m_v, idx_v, x_v, sem_x, sem_ym, sem_g, sem_out):
    wid = lax.axis_index("s") * 2 + lax.axis_index("c")
    nch = (_NCHUNKS - wid + _NW - 1) // _NW

    def chunk_base(c):
        return (wid + c * _NW) * _C

    def fire_in(c, b):
        base = chunk_base(c)
        pltpu.async_copy(x_hbm.at[pl.ds(base, _C)], x_v.at[b], sem_x.at[b])
        pltpu.async_copy(y_hbm.at[pl.ds(base, _C)],
                         y_v.at[pl.ds(b * _C, _C)], sem_ym.at[b])
        pltpu.async_copy(m_hbm.at[pl.ds(base, _C)],
                         m_v.at[pl.ds(b * _C, _C)], sem_ym.at[b])

    def wait_out(b):
        pltpu.make_async_copy(
            x_v.at[b], out_hbm.at[pl.ds(0, _C)], sem_out.at[b]).wait()

    def mid(c, b):
        pltpu.make_async_copy(
            y_hbm.at[pl.ds(0, _C)], y_v.at[pl.ds(b * _C, _C)],
            sem_ym.at[b]).wait()
        pltpu.make_async_copy(
            m_hbm.at[pl.ds(0, _C)], m_v.at[pl.ds(b * _C, _C)],
            sem_ym.at[b]).wait()

        def sel_body(g, c2):
            s = pl.ds(b * _C + g * _L, _L)
            yv = y_v[s]
            idx_v[s] = jnp.where(m_v[s] != 0, yv, yv + _NUM_CLASSES)
            return c2

        lax.fori_loop(0, _C // _L, sel_body, 0)

        pltpu.make_async_copy(
            x_hbm.at[pl.ds(0, _C)], x_v.at[b], sem_x.at[b]).wait()
        for j in range(_C // _G):
            pltpu.async_copy(
                tab_hbm.at[idx_v.at[pl.ds(b * _C + j * _G, _G)]],
                x_v.at[b].at[pl.ds(j * _G, _G)], sem_g.at[b], add=True)

    def finish(c, b):
        base = chunk_base(c)
        for j in range(_C // _G):
            pltpu.make_async_copy(
                tab_hbm.at[idx_v.at[pl.ds(b * _C + j * _G, _G)]],
                x_v.at[b].at[pl.ds(j * _G, _G)], sem_g.at[b]).wait()
        pltpu.async_copy(x_v.at[b], out_hbm.at[pl.ds(base, _C)],
                         sem_out.at[b])

    for b in range(_NB):
        fire_in(b, b)

    def group_body(k4, carry):
        c0 = k4 * _NB
        for b in range(_NB):
            c = c0 + b

            @pl.when(c < nch)
            def _():
                mid(c, b)
        for b in range(_NB):
            c = c0 + b

            @pl.when(c < nch)
            def _():
                finish(c, b)
        for b in range(_NB):
            c = c0 + b + _NB

            @pl.when(c < nch)
            def _():
                wait_out(b)
                fire_in(c, b)
        return carry

    lax.fori_loop(0, _NGROUPS, group_body, 0)

    for b in range(_NB):
        wait_out(b)


def kernel(x, y, mask, emb_weight):
    table = jnp.concatenate(
        [emb_weight, jnp.zeros((_NUM_CLASSES, _D), jnp.float32)], axis=0)
    return _mask_label_sc(x, y, mask.astype(jnp.int32), table)


# speedup vs baseline: 1.6951x; 1.2786x over previous
import functools

import jax
import jax.numpy as jnp
from jax import lax
from jax.experimental import pallas as pl
from jax.experimental.pallas import tpu as pltpu
from jax.experimental.pallas import tpu_sc as plsc

_N = 100000
_D = 128
_NUM_CLASSES = 1000
_C = 160
_G = 80
_NCHUNKS = _N // _C
_NW = 32
_NB = 4
_L = 16
_NGROUPS = (_NCHUNKS // _NW + _NB) // _NB

_mesh = plsc.VectorSubcoreMesh(core_axis_name="c", subcore_axis_name="s")


@functools.partial(
    pl.kernel,
    mesh=_mesh,
    out_type=jax.ShapeDtypeStruct((_N, _D), jnp.float32),
    scratch_types=[
        pltpu.VMEM((_NB * _C,), jnp.int32),
        pltpu.VMEM((_NB * _C,), jnp.int32),
        pltpu.VMEM((_NB * _C,), jnp.int32),
        pltpu.VMEM((_NB, _C, _D), jnp.float32),
        pltpu.VMEM_SHARED((2 * _NUM_CLASSES, _D), jnp.float32),
        pltpu.SemaphoreType.DMA((_NB,)),
        pltpu.SemaphoreType.DMA((_NB,)),
        pltpu.SemaphoreType.DMA((_NB,)),
        pltpu.SemaphoreType.DMA((_NB,)),
    ],
)
def _mask_label_sc(x_hbm, y_hbm, m_hbm, tab_hbm, out_hbm,
                   y_v, m_v, idx_v, x_v, tab_sh,
                   sem_x, sem_ym, sem_g, sem_out):
    sid = lax.axis_index("s")
    wid = sid * 2 + lax.axis_index("c")
    nch = (_NCHUNKS - wid + _NW - 1) // _NW

    @pl.when(sid == 0)
    def _():
        pltpu.sync_copy(tab_hbm, tab_sh)

    plsc.subcore_barrier()

    def chunk_base(c):
        return (wid + c * _NW) * _C

    def fire_in(c, b):
        base = chunk_base(c)
        pltpu.async_copy(x_hbm.at[pl.ds(base, _C)], x_v.at[b], sem_x.at[b])
        pltpu.async_copy(y_hbm.at[pl.ds(base, _C)],
                         y_v.at[pl.ds(b * _C, _C)], sem_ym.at[b])
        pltpu.async_copy(m_hbm.at[pl.ds(base, _C)],
                         m_v.at[pl.ds(b * _C, _C)], sem_ym.at[b])

    def wait_out(b):
        pltpu.make_async_copy(
            x_v.at[b], out_hbm.at[pl.ds(0, _C)], sem_out.at[b]).wait()

    def mid(c, b):
        pltpu.make_async_copy(
            y_hbm.at[pl.ds(0, _C)], y_v.at[pl.ds(b * _C, _C)],
            sem_ym.at[b]).wait()
        pltpu.make_async_copy(
            m_hbm.at[pl.ds(0, _C)], m_v.at[pl.ds(b * _C, _C)],
            sem_ym.at[b]).wait()

        def sel_body(g, c2):
            s = pl.ds(b * _C + g * _L, _L)
            yv = y_v[s]
            idx_v[s] = jnp.where(m_v[s] != 0, yv, yv + _NUM_CLASSES)
            return c2

        lax.fori_loop(0, _C // _L, sel_body, 0)

        pltpu.make_async_copy(
            x_hbm.at[pl.ds(0, _C)], x_v.at[b], sem_x.at[b]).wait()
        for j in range(_C // _G):
            pltpu.async_copy(
                tab_sh.at[idx_v.at[pl.ds(b * _C + j * _G, _G)]],
                x_v.at[b].at[pl.ds(j * _G, _G)], sem_g.at[b], add=True)

    def finish(c, b):
        base = chunk_base(c)
        for j in range(_C // _G):
            pltpu.make_async_copy(
                tab_sh.at[idx_v.at[pl.ds(b * _C + j * _G, _G)]],
                x_v.at[b].at[pl.ds(j * _G, _G)], sem_g.at[b]).wait()
        pltpu.async_copy(x_v.at[b], out_hbm.at[pl.ds(base, _C)],
                         sem_out.at[b])

    for b in range(_NB):
        fire_in(b, b)

    def group_body(k4, carry):
        c0 = k4 * _NB
        for b in range(_NB):
            c = c0 + b

            @pl.when(c < nch)
            def _():
                mid(c, b)
        for b in range(_NB):
            c = c0 + b

            @pl.when(c < nch)
            def _():
                finish(c, b)
        for b in range(_NB):
            c = c0 + b + _NB

            @pl.when(c < nch)
            def _():
                wait_out(b)
                fire_in(c, b)
        return carry

    lax.fori_loop(0, _NGROUPS, group_body, 0)

    for b in range(_NB):
        wait_out(b)


def kernel(x, y, mask, emb_weight):
    table = jnp.concatenate(
        [emb_weight, jnp.zeros((_NUM_CLASSES, _D), jnp.float32)], axis=0)
    return _mask_label_sc(x, y, mask.astype(jnp.int32), table)


# speedup vs baseline: 1.7358x; 1.0240x over previous
import functools

import jax
import jax.numpy as jnp
from jax import lax
from jax.experimental import pallas as pl
from jax.experimental.pallas import tpu as pltpu
from jax.experimental.pallas import tpu_sc as plsc

_N = 100000
_D = 128
_NUM_CLASSES = 1000
_C = 160
_G = 80
_NCHUNKS = _N // _C
_NW = 32
_NB = 5
_L = 16
_NGROUPS = (_NCHUNKS // _NW + _NB) // _NB
_TROWS = 2048

_mesh = plsc.VectorSubcoreMesh(core_axis_name="c", subcore_axis_name="s")


@functools.partial(
    pl.kernel,
    mesh=_mesh,
    out_type=jax.ShapeDtypeStruct((_N, _D), jnp.float32),
    scratch_types=[
        pltpu.VMEM((_NB * _C,), jnp.int32),
        pltpu.VMEM((_NB * _C,), jnp.int32),
        pltpu.VMEM((_NB * _C,), jnp.int32),
        pltpu.VMEM((_NB, _C, _D), jnp.float32),
        pltpu.VMEM_SHARED((_TROWS, _D), jnp.float32),
        pltpu.SemaphoreType.DMA((_NB,)),
        pltpu.SemaphoreType.DMA((_NB,)),
        pltpu.SemaphoreType.DMA((_NB,)),
        pltpu.SemaphoreType.DMA((_NB,)),
    ],
)
def _mask_label_sc(x_hbm, y_hbm, m_hbm, tab_hbm, out_hbm,
                   y_v, m_v, idx_v, x_v, tab_sh,
                   sem_x, sem_ym, sem_g, sem_out):
    sid = lax.axis_index("s")
    wid = sid * 2 + lax.axis_index("c")
    nch = (_NCHUNKS - wid + _NW - 1) // _NW

    def chunk_base(c):
        return (wid + c * _NW) * _C

    def fire_in(c, b):
        base = chunk_base(c)
        pltpu.async_copy(x_hbm.at[pl.ds(base, _C)], x_v.at[b], sem_x.at[b])
        pltpu.async_copy(y_hbm.at[pl.ds(base, _C)],
                         y_v.at[pl.ds(b * _C, _C)], sem_ym.at[b])
        pltpu.async_copy(m_hbm.at[pl.ds(base, _C)],
                         m_v.at[pl.ds(b * _C, _C)], sem_ym.at[b])

    def wait_out(b):
        pltpu.make_async_copy(
            x_v.at[b], out_hbm.at[pl.ds(0, _C)], sem_out.at[b]).wait()

    def mid(c, b):
        pltpu.make_async_copy(
            y_hbm.at[pl.ds(0, _C)], y_v.at[pl.ds(b * _C, _C)],
            sem_ym.at[b]).wait()
        pltpu.make_async_copy(
            m_hbm.at[pl.ds(0, _C)], m_v.at[pl.ds(b * _C, _C)],
            sem_ym.at[b]).wait()

        def sel_body(g, c2):
            s = pl.ds(b * _C + g * _L, _L)
            yv = y_v[s]
            idx_v[s] = jnp.where(m_v[s] != 0, yv, yv + _NUM_CLASSES)
            return c2

        lax.fori_loop(0, _C // _L, sel_body, 0)

        pltpu.make_async_copy(
            x_hbm.at[pl.ds(0, _C)], x_v.at[b], sem_x.at[b]).wait()
        for j in range(_C // _G):
            pltpu.async_copy(
                tab_sh.at[idx_v.at[pl.ds(b * _C + j * _G, _G)]],
                x_v.at[b].at[pl.ds(j * _G, _G)], sem_g.at[b], add=True)

    def finish(c, b):
        base = chunk_base(c)
        for j in range(_C // _G):
            pltpu.make_async_copy(
                tab_sh.at[idx_v.at[pl.ds(b * _C + j * _G, _G)]],
                x_v.at[b].at[pl.ds(j * _G, _G)], sem_g.at[b]).wait()
        pltpu.async_copy(x_v.at[b], out_hbm.at[pl.ds(base, _C)],
                         sem_out.at[b])

    for b in range(_NB):
        fire_in(b, b)

    _TR = _TROWS // 16
    pltpu.sync_copy(tab_hbm.at[pl.ds(sid * _TR, _TR)],
                    tab_sh.at[pl.ds(sid * _TR, _TR)])
    plsc.subcore_barrier()

    def group_body(k4, carry):
        c0 = k4 * _NB
        for b in range(_NB):
            c = c0 + b

            @pl.when(c < nch)
            def _():
                mid(c, b)
        for b in range(_NB):
            c = c0 + b

            @pl.when(c < nch)
            def _():
                finish(c, b)
        for b in range(_NB):
            c = c0 + b + _NB

            @pl.when(c < nch)
            def _():
                wait_out(b)
                fire_in(c, b)
        return carry

    lax.fori_loop(0, _NGROUPS, group_body, 0)

    for b in range(_NB):
        wait_out(b)


def kernel(x, y, mask, emb_weight):
    table = jnp.concatenate(
        [emb_weight, jnp.zeros((_TROWS - _NUM_CLASSES, _D), jnp.float32)],
        axis=0)
    return _mask_label_sc(x, y, mask.astype(jnp.int32), table)
